# baseline (speedup 1.0000x reference)
import dataclasses
import functools

import jax
import jax.numpy as jnp
from jax import lax
from jax.experimental import pallas as pl
from jax.experimental.pallas import tpu as pltpu
from jax.experimental.pallas import tpu_sc as plsc

N = 10000
D = 128
E = 320000
NC = 2
NS = 16
L = 16
NW = NC * NS
EW = E // NW
K = 64
G = 8
C = 160
EWP = C * K
NP = 10016
NT = NP
RPS = 624

BLK = 1000


def _logit_tables(h, as_ref, ad_ref, av_ref, bv_ref, m_ref):
    av = jnp.dot(h, as_ref[...], preferred_element_type=jnp.float32)
    bv = jnp.dot(h, ad_ref[...], preferred_element_type=jnp.float32)
    av_ref[...] = av
    bv_ref[...] = bv
    pm = jnp.concatenate(
        [jnp.max(av).reshape(1, 1), jnp.max(bv).reshape(1, 1)], axis=1)

    @pl.when(pl.program_id(0) == 0)
    def _():
        m_ref[...] = pm

    @pl.when(pl.program_id(0) != 0)
    def _():
        m_ref[...] = jnp.maximum(m_ref[...], pm)


def _tc_in_body(x_ref, w_ref, as_ref, ad_ref, h_ref, av_ref, bv_ref, m_ref):
    h = jnp.dot(x_ref[...], w_ref[...], preferred_element_type=jnp.float32)
    h_ref[...] = h
    _logit_tables(h, as_ref, ad_ref, av_ref, bv_ref, m_ref)


_DENSE_OUT_SPECS = [
    pl.BlockSpec((BLK, D), lambda i: (i, 0)),
    pl.BlockSpec((BLK, 1), lambda i: (i, 0)),
    pl.BlockSpec((BLK, 1), lambda i: (i, 0)),
    pl.BlockSpec((1, 2), lambda i: (0, 0)),
]
_DENSE_OUT_SHAPE = [
    jax.ShapeDtypeStruct((N, D), jnp.float32),
    jax.ShapeDtypeStruct((N, 1), jnp.float32),
    jax.ShapeDtypeStruct((N, 1), jnp.float32),
    jax.ShapeDtypeStruct((1, 2), jnp.float32),
]


def _tc_in(x, W, a_src, a_dst):
    return pl.pallas_call(
        _tc_in_body,
        grid=(N // BLK,),
        in_specs=[
            pl.BlockSpec((BLK, D), lambda i: (i, 0)),
            pl.BlockSpec((D, D), lambda i: (0, 0)),
            pl.BlockSpec((D, 1), lambda i: (0, 0)),
            pl.BlockSpec((D, 1), lambda i: (0, 0)),
        ],
        out_specs=_DENSE_OUT_SPECS,
        out_shape=_DENSE_OUT_SHAPE,
    )(x, W, a_src, a_dst)


def _tc_mid_body(r0_ref, r1_ref, u_ref, b_ref, w_ref, as_ref, ad_ref,
                 h_ref, av_ref, bv_ref, m_ref):
    num = r0_ref[...] + r1_ref[...]
    den = jnp.sum(u_ref[...], axis=1, keepdims=True)
    z = num / (den + 1e-16) + b_ref[...]
    z = jnp.maximum(z, 0.0)
    h = jnp.dot(z, w_ref[...], preferred_element_type=jnp.float32)
    h_ref[...] = h
    _logit_tables(h, as_ref, ad_ref, av_ref, bv_ref, m_ref)


def _tc_mid(r0, r1, u, b, W, a_src, a_dst):
    return pl.pallas_call(
        _tc_mid_body,
        grid=(N // BLK,),
        in_specs=[
            pl.BlockSpec((BLK, D), lambda i: (i, 0)),
            pl.BlockSpec((BLK, D), lambda i: (i, 0)),
            pl.BlockSpec((BLK, NW), lambda i: (i, 0)),
            pl.BlockSpec((1, D), lambda i: (0, 0)),
            pl.BlockSpec((D, D), lambda i: (0, 0)),
            pl.BlockSpec((D, 1), lambda i: (0, 0)),
            pl.BlockSpec((D, 1), lambda i: (0, 0)),
        ],
        out_specs=_DENSE_OUT_SPECS,
        out_shape=_DENSE_OUT_SHAPE,
    )(r0, r1, u, b, W, a_src, a_dst)


def _tc_out_body(r0_ref, r1_ref, u_ref, b_ref, o_ref):
    num = r0_ref[...] + r1_ref[...]
    den = jnp.sum(u_ref[...], axis=1, keepdims=True)
    o_ref[...] = num / (den + 1e-16) + b_ref[...]


def _tc_out(r0, r1, u, b):
    return pl.pallas_call(
        _tc_out_body,
        grid=(N // BLK,),
        in_specs=[
            pl.BlockSpec((BLK, D), lambda i: (i, 0)),
            pl.BlockSpec((BLK, D), lambda i: (i, 0)),
            pl.BlockSpec((BLK, NW), lambda i: (i, 0)),
            pl.BlockSpec((1, D), lambda i: (0, 0)),
        ],
        out_specs=pl.BlockSpec((BLK, D), lambda i: (i, 0)),
        out_shape=jax.ShapeDtypeStruct((N, D), jnp.float32),
    )(r0, r1, u, b)


_SC_PARAMS = pltpu.CompilerParams()
if "needs_layout_passes" in pltpu.CompilerParams.__dataclass_fields__:
    _SC_PARAMS = dataclasses.replace(_SC_PARAMS, needs_layout_passes=False)


def _edge_pass_body(h_hbm, ast_hbm, adt_hbm, src_hbm, dst_hbm, m_hbm,
                    z128_hbm, rows_out, den_out, src_v, dst_v, as_v, ad_v,
                    m_v, rows_a, rows_b, den_v, acc_rows, sem_a, sem_b,
                    sem_sa, sem_sb):
    cid = lax.axis_index("c")
    sid = lax.axis_index("s")
    wid = cid * NS + sid

    pltpu.sync_copy(z128_hbm.at[pl.ds(0, RPS)],
                    acc_rows.at[pl.ds(sid * RPS, RPS)])

    @pl.when(sid == 0)
    def _():
        tail = NP - NS * RPS
        pltpu.sync_copy(z128_hbm.at[pl.ds(0, tail)],
                        acc_rows.at[pl.ds(NS * RPS, tail)])

    pltpu.sync_copy(ast_hbm, as_v)
    pltpu.sync_copy(adt_hbm, ad_v)
    pltpu.sync_copy(m_hbm, m_v)

    zero16 = jnp.zeros((L,), jnp.float32)

    @pl.loop(0, NT // L)
    def _zero(i):
        den_v[pl.ds(i * L, L)] = zero16

    plsc.subcore_barrier()

    mvec = m_v[...]
    masks = [lax.iota(jnp.int32, L) == t for t in range(L)]

    def _process(g, rows_v):
        for l in range(K // L):
            s16 = src_v[g, pl.ds(l * L, L)]
            d16 = dst_v[g, pl.ds(l * L, L)]
            e = plsc.load_gather(as_v, [s16]) + plsc.load_gather(ad_v, [d16])
            e = jnp.maximum(e, 0.2 * e)
            w = jnp.exp(e - mvec)
            for t in range(L):
                plsc.addupdate_scatter(den_v, [d16], w, mask=masks[t])
                r = l * L + t
                wb = jnp.broadcast_to(w[t], (L,))
                for c in range(D // L):
                    sl = pl.ds(c * L, L)
                    rows_v[r, sl] = rows_v[r, sl] * wb

    @pl.loop(0, C // G)
    def _super(o):
        pltpu.sync_copy(src_hbm.at[wid].at[pl.ds(o * G, G)], src_v)
        pltpu.sync_copy(dst_hbm.at[wid].at[pl.ds(o * G, G)], dst_v)
        pltpu.async_copy(h_hbm.at[src_v.at[0]], rows_a, sem_a)

        @pl.loop(0, G // 2)
        def _pair(p):
            g0 = 2 * p

            @pl.when(p > 0)
            def _():
                pltpu.make_async_copy(
                    rows_b, acc_rows.at[dst_v.at[g0 - 1]], sem_sb).wait()

            pltpu.async_copy(h_hbm.at[src_v.at[g0 + 1]], rows_b, sem_b)
            pltpu.make_async_copy(h_hbm.at[src_v.at[g0]], rows_a, sem_a).wait()
            _process(g0, rows_a)
            pltpu.async_copy(rows_a, acc_rows.at[dst_v.at[g0]], sem_sa,
                             add=True)
            pltpu.make_async_copy(h_hbm.at[src_v.at[g0 + 1]], rows_b,
                                  sem_b).wait()
            _process(g0 + 1, rows_b)
            pltpu.make_async_copy(
                rows_a, acc_rows.at[dst_v.at[g0]], sem_sa).wait()

            @pl.when(p < G // 2 - 1)
            def _():
                pltpu.async_copy(h_hbm.at[src_v.at[g0 + 2]], rows_a, sem_a)

            pltpu.async_copy(rows_b, acc_rows.at[dst_v.at[g0 + 1]], sem_sb,
                             add=True)

        pltpu.make_async_copy(rows_b, acc_rows.at[dst_v.at[G - 1]],
                              sem_sb).wait()

    plsc.subcore_barrier()
    sl = pl.ds(sid * RPS, RPS)
    pltpu.sync_copy(acc_rows.at[sl], rows_out.at[cid].at[sl])

    @pl.when(sid == 0)
    def _():
        tl = pl.ds(NS * RPS, N - NS * RPS)
        pltpu.sync_copy(acc_rows.at[tl], rows_out.at[cid].at[tl])

    pltpu.sync_copy(den_v, den_out.at[wid])


@functools.lru_cache(maxsize=1)
def _edge_pass():
    mesh = plsc.VectorSubcoreMesh(core_axis_name="c", subcore_axis_name="s",
                                  num_cores=NC, num_subcores=NS)
    return functools.partial(
        pl.kernel,
        out_type=(
            jax.ShapeDtypeStruct((NC, N, D), jnp.float32),
            jax.ShapeDtypeStruct((NW, NT), jnp.float32),
        ),
        mesh=mesh,
        compiler_params=_SC_PARAMS,
        scratch_types=[
            pltpu.VMEM((G, K), jnp.int32),
            pltpu.VMEM((G, K), jnp.int32),
            pltpu.VMEM((NT,), jnp.float32),
            pltpu.VMEM((NT,), jnp.float32),
            pltpu.VMEM((L,), jnp.float32),
            pltpu.VMEM((K, D), jnp.float32),
            pltpu.VMEM((K, D), jnp.float32),
            pltpu.VMEM((NT,), jnp.float32),
            pltpu.VMEM_SHARED((NP, D), jnp.float32),
            pltpu.SemaphoreType.DMA,
            pltpu.SemaphoreType.DMA,
            pltpu.SemaphoreType.DMA,
            pltpu.SemaphoreType.DMA,
        ],
    )(_edge_pass_body)


def _pad_tab(col):
    return jnp.pad(col.reshape(N), (0, NT - N))


def _pad_idx(row, fill):
    r = row.astype(jnp.int32).reshape(NW, EW)
    r = jnp.pad(r, ((0, 0), (0, EWP - EW)), constant_values=fill)
    return r.reshape(NW, C, K)


def kernel(x, edge_index, W1, a1_src, a1_dst, b1, W2, a2_src, a2_dst, b2):
    src3 = _pad_idx(edge_index[0], 0)
    dst3 = _pad_idx(edge_index[1], N)
    z128 = jnp.zeros((RPS, D), jnp.float32)
    edge = _edge_pass()

    h1, as1, ad1, m1 = _tc_in(x, W1, a1_src.reshape(D, 1),
                              a1_dst.reshape(D, 1))
    M1 = jnp.maximum(m1[0, 0] + m1[0, 1], 0.0)
    r1, u1 = edge(h1, _pad_tab(as1), _pad_tab(ad1), src3, dst3,
                  jnp.broadcast_to(M1, (L,)), z128)

    h2, as2, ad2, m2 = _tc_mid(r1[0], r1[1], u1.T[:N], b1.reshape(1, D),
                               W2, a2_src.reshape(D, 1), a2_dst.reshape(D, 1))
    M2 = jnp.maximum(m2[0, 0] + m2[0, 1], 0.0)
    r2, u2 = edge(h2, _pad_tab(as2), _pad_tab(ad2), src3, dst3,
                  jnp.broadcast_to(M2, (L,)), z128)

    return _tc_out(r2[0], r2[1], u2.T[:N], b2.reshape(1, D))


# speedup vs baseline: 1.0154x; 1.0154x over previous
import dataclasses
import functools

import jax
import jax.numpy as jnp
from jax import lax
from jax.experimental import pallas as pl
from jax.experimental.pallas import tpu as pltpu
from jax.experimental.pallas import tpu_sc as plsc

N = 10000
D = 128
E = 320000
NC = 2
NS = 16
L = 16
NW = NC * NS
EW = E // NW
K = 64
G = 8
C = 160
EWP = C * K
NP = 10016
NT = NP
RPS = 624

BLK = 1000


def _logit_tables(h, as_ref, ad_ref, av_ref, bv_ref, m_ref):
    av = jnp.dot(h, as_ref[...], preferred_element_type=jnp.float32)
    bv = jnp.dot(h, ad_ref[...], preferred_element_type=jnp.float32)
    av_ref[...] = av
    bv_ref[...] = bv
    pm = jnp.concatenate(
        [jnp.max(av).reshape(1, 1), jnp.max(bv).reshape(1, 1)], axis=1)

    @pl.when(pl.program_id(0) == 0)
    def _():
        m_ref[...] = pm

    @pl.when(pl.program_id(0) != 0)
    def _():
        m_ref[...] = jnp.maximum(m_ref[...], pm)


def _tc_in_body(x_ref, w_ref, as_ref, ad_ref, h_ref, av_ref, bv_ref, m_ref):
    h = jnp.dot(x_ref[...], w_ref[...], preferred_element_type=jnp.float32)
    h_ref[...] = h
    _logit_tables(h, as_ref, ad_ref, av_ref, bv_ref, m_ref)


_DENSE_OUT_SPECS = [
    pl.BlockSpec((BLK, D), lambda i: (i, 0)),
    pl.BlockSpec((BLK, 1), lambda i: (i, 0)),
    pl.BlockSpec((BLK, 1), lambda i: (i, 0)),
    pl.BlockSpec((1, 2), lambda i: (0, 0)),
]
_DENSE_OUT_SHAPE = [
    jax.ShapeDtypeStruct((N, D), jnp.float32),
    jax.ShapeDtypeStruct((N, 1), jnp.float32),
    jax.ShapeDtypeStruct((N, 1), jnp.float32),
    jax.ShapeDtypeStruct((1, 2), jnp.float32),
]


def _tc_in(x, W, a_src, a_dst):
    return pl.pallas_call(
        _tc_in_body,
        grid=(N // BLK,),
        in_specs=[
            pl.BlockSpec((BLK, D), lambda i: (i, 0)),
            pl.BlockSpec((D, D), lambda i: (0, 0)),
            pl.BlockSpec((D, 1), lambda i: (0, 0)),
            pl.BlockSpec((D, 1), lambda i: (0, 0)),
        ],
        out_specs=_DENSE_OUT_SPECS,
        out_shape=_DENSE_OUT_SHAPE,
    )(x, W, a_src, a_dst)


def _tc_mid_body(r0_ref, r1_ref, u_ref, b_ref, w_ref, as_ref, ad_ref,
                 h_ref, av_ref, bv_ref, m_ref):
    num = r0_ref[...] + r1_ref[...]
    den = jnp.sum(u_ref[...], axis=1, keepdims=True)
    z = num / (den + 1e-16) + b_ref[...]
    z = jnp.maximum(z, 0.0)
    h = jnp.dot(z, w_ref[...], preferred_element_type=jnp.float32)
    h_ref[...] = h
    _logit_tables(h, as_ref, ad_ref, av_ref, bv_ref, m_ref)


def _tc_mid(r0, r1, u, b, W, a_src, a_dst):
    return pl.pallas_call(
        _tc_mid_body,
        grid=(N // BLK,),
        in_specs=[
            pl.BlockSpec((BLK, D), lambda i: (i, 0)),
            pl.BlockSpec((BLK, D), lambda i: (i, 0)),
            pl.BlockSpec((BLK, NW), lambda i: (i, 0)),
            pl.BlockSpec((1, D), lambda i: (0, 0)),
            pl.BlockSpec((D, D), lambda i: (0, 0)),
            pl.BlockSpec((D, 1), lambda i: (0, 0)),
            pl.BlockSpec((D, 1), lambda i: (0, 0)),
        ],
        out_specs=_DENSE_OUT_SPECS,
        out_shape=_DENSE_OUT_SHAPE,
    )(r0, r1, u, b, W, a_src, a_dst)


def _tc_out_body(r0_ref, r1_ref, u_ref, b_ref, o_ref):
    num = r0_ref[...] + r1_ref[...]
    den = jnp.sum(u_ref[...], axis=1, keepdims=True)
    o_ref[...] = num / (den + 1e-16) + b_ref[...]


def _tc_out(r0, r1, u, b):
    return pl.pallas_call(
        _tc_out_body,
        grid=(N // BLK,),
        in_specs=[
            pl.BlockSpec((BLK, D), lambda i: (i, 0)),
            pl.BlockSpec((BLK, D), lambda i: (i, 0)),
            pl.BlockSpec((BLK, NW), lambda i: (i, 0)),
            pl.BlockSpec((1, D), lambda i: (0, 0)),
        ],
        out_specs=pl.BlockSpec((BLK, D), lambda i: (i, 0)),
        out_shape=jax.ShapeDtypeStruct((N, D), jnp.float32),
    )(r0, r1, u, b)


_SC_PARAMS = pltpu.CompilerParams()
if "needs_layout_passes" in pltpu.CompilerParams.__dataclass_fields__:
    _SC_PARAMS = dataclasses.replace(_SC_PARAMS, needs_layout_passes=False)


def _edge_pass_body(h_hbm, ast_hbm, adt_hbm, src_hbm, dst_hbm, m_hbm,
                    z128_hbm, rows_out, den_out, src_v, dst_v, as_v, ad_v,
                    m_v, rows_a, rows_b, den_v, acc_rows, sem_a, sem_b):
    cid = lax.axis_index("c")
    sid = lax.axis_index("s")
    wid = cid * NS + sid

    pltpu.sync_copy(z128_hbm.at[pl.ds(0, RPS)],
                    acc_rows.at[pl.ds(sid * RPS, RPS)])

    @pl.when(sid == 0)
    def _():
        tail = NP - NS * RPS
        pltpu.sync_copy(z128_hbm.at[pl.ds(0, tail)],
                        acc_rows.at[pl.ds(NS * RPS, tail)])

    pltpu.sync_copy(ast_hbm, as_v)
    pltpu.sync_copy(adt_hbm, ad_v)
    pltpu.sync_copy(m_hbm, m_v)

    zero16 = jnp.zeros((L,), jnp.float32)

    @pl.loop(0, NT // L)
    def _zero(i):
        den_v[pl.ds(i * L, L)] = zero16

    plsc.subcore_barrier()

    mvec = m_v[...]
    masks = [lax.iota(jnp.int32, L) == t for t in range(L)]

    def _process(g, rows_v):
        for l in range(K // L):
            s16 = src_v[g, pl.ds(l * L, L)]
            d16 = dst_v[g, pl.ds(l * L, L)]
            e = plsc.load_gather(as_v, [s16]) + plsc.load_gather(ad_v, [d16])
            e = jnp.maximum(e, 0.2 * e)
            w = jnp.exp(e - mvec)
            for t in range(L):
                plsc.addupdate_scatter(den_v, [d16], w, mask=masks[t])
                r = l * L + t
                wb = jnp.broadcast_to(w[t], (L,))
                for c in range(D // L):
                    sl = pl.ds(c * L, L)
                    rows_v[r, sl] = rows_v[r, sl] * wb
        pltpu.sync_copy(rows_v, acc_rows.at[dst_v.at[g]], add=True)

    @pl.loop(0, C // G)
    def _super(o):
        pltpu.sync_copy(src_hbm.at[wid].at[pl.ds(o * G, G)], src_v)
        pltpu.sync_copy(dst_hbm.at[wid].at[pl.ds(o * G, G)], dst_v)
        pltpu.async_copy(h_hbm.at[src_v.at[0]], rows_a, sem_a)

        @pl.loop(0, G // 2)
        def _pair(p):
            g0 = 2 * p
            pltpu.async_copy(h_hbm.at[src_v.at[g0 + 1]], rows_b, sem_b)
            pltpu.make_async_copy(h_hbm.at[src_v.at[g0]], rows_a, sem_a).wait()
            _process(g0, rows_a)

            @pl.when(p < G // 2 - 1)
            def _():
                pltpu.async_copy(h_hbm.at[src_v.at[g0 + 2]], rows_a, sem_a)

            pltpu.make_async_copy(h_hbm.at[src_v.at[g0 + 1]], rows_b,
                                  sem_b).wait()
            _process(g0 + 1, rows_b)

    plsc.subcore_barrier()
    sl = pl.ds(sid * RPS, RPS)
    pltpu.sync_copy(acc_rows.at[sl], rows_out.at[cid].at[sl])

    @pl.when(sid == 0)
    def _():
        tl = pl.ds(NS * RPS, N - NS * RPS)
        pltpu.sync_copy(acc_rows.at[tl], rows_out.at[cid].at[tl])

    pltpu.sync_copy(den_v, den_out.at[wid])


@functools.lru_cache(maxsize=1)
def _edge_pass():
    mesh = plsc.VectorSubcoreMesh(core_axis_name="c", subcore_axis_name="s",
                                  num_cores=NC, num_subcores=NS)
    return functools.partial(
        pl.kernel,
        out_type=(
            jax.ShapeDtypeStruct((NC, N, D), jnp.float32),
            jax.ShapeDtypeStruct((NW, NT), jnp.float32),
        ),
        mesh=mesh,
        compiler_params=_SC_PARAMS,
        scratch_types=[
            pltpu.VMEM((G, K), jnp.int32),
            pltpu.VMEM((G, K), jnp.int32),
            pltpu.VMEM((NT,), jnp.float32),
            pltpu.VMEM((NT,), jnp.float32),
            pltpu.VMEM((L,), jnp.float32),
            pltpu.VMEM((K, D), jnp.float32),
            pltpu.VMEM((K, D), jnp.float32),
            pltpu.VMEM((NT,), jnp.float32),
            pltpu.VMEM_SHARED((NP, D), jnp.float32),
            pltpu.SemaphoreType.DMA,
            pltpu.SemaphoreType.DMA,
        ],
    )(_edge_pass_body)


def _pad_tab(col):
    return jnp.pad(col.reshape(N), (0, NT - N))


def _pad_idx(row, fill):
    r = row.astype(jnp.int32).reshape(NW, EW)
    r = jnp.pad(r, ((0, 0), (0, EWP - EW)), constant_values=fill)
    return r.reshape(NW, C, K)


def kernel(x, edge_index, W1, a1_src, a1_dst, b1, W2, a2_src, a2_dst, b2):
    src3 = _pad_idx(edge_index[0], 0)
    dst3 = _pad_idx(edge_index[1], N)
    z128 = jnp.zeros((RPS, D), jnp.float32)
    edge = _edge_pass()

    h1, as1, ad1, m1 = _tc_in(x, W1, a1_src.reshape(D, 1),
                              a1_dst.reshape(D, 1))
    M1 = jnp.maximum(m1[0, 0] + m1[0, 1], 0.0)
    r1, u1 = edge(h1, _pad_tab(as1), _pad_tab(ad1), src3, dst3,
                  jnp.broadcast_to(M1, (L,)), z128)

    h2, as2, ad2, m2 = _tc_mid(r1[0], r1[1], u1.T[:N], b1.reshape(1, D),
                               W2, a2_src.reshape(D, 1), a2_dst.reshape(D, 1))
    M2 = jnp.maximum(m2[0, 0] + m2[0, 1], 0.0)
    r2, u2 = edge(h2, _pad_tab(as2), _pad_tab(ad2), src3, dst3,
                  jnp.broadcast_to(M2, (L,)), z128)

    return _tc_out(r2[0], r2[1], u2.T[:N], b2.reshape(1, D))
